# Initial kernel scaffold; baseline (speedup 1.0000x reference)
import functools

import jax
import jax.numpy as jnp
from jax import lax
from jax.experimental import pallas as pl
from jax.experimental.pallas import tpu as pltpu
from jax.experimental.pallas import tpu_sc as plsc

N_NODES = 100000
N_EDGES = 3200000
HID = 64
FB = 16
NFB = HID // FB
NC = 2
NS = 16
CHUNK = 128
NCHUNKS = N_EDGES // CHUNK
CH_BASE = NCHUNKS // NS
CH_REM = NCHUNKS - CH_BASE * NS
ROWS_PER_SUB = N_NODES // NS
ZROWS = 625

NBLK = 2000
NGRID = N_NODES // NBLK


def _leaky(x):
    return jnp.where(x >= 0, x, 0.01 * x)


def _sc_agg_body(edge_hbm, h0, h1, h2, h3, a0, a1, a2, a3,
                 idx_buf, rows_buf, zero_buf, acc):
    c = lax.axis_index("c")
    s = lax.axis_index("s")
    start_chunk = s * CH_BASE + jnp.minimum(s, CH_REM)
    count = CH_BASE + jnp.where(s < CH_REM, 1, 0)
    row0 = s * ROWS_PER_SUB

    @pl.loop(0, ZROWS)
    def _(i):
        zero_buf[i, :] = jnp.zeros((FB,), jnp.float32)

    def one_pass(table_ref, out_ref):
        @pl.loop(0, ROWS_PER_SUB // ZROWS)
        def _(i):
            pltpu.sync_copy(zero_buf, acc.at[pl.ds(row0 + i * ZROWS, ZROWS)])
        plsc.subcore_barrier()

        @pl.loop(0, count)
        def _(i):
            base = (start_chunk + i) * CHUNK
            pltpu.sync_copy(edge_hbm.at[0, pl.ds(base, CHUNK)], idx_buf.at[0])
            pltpu.sync_copy(edge_hbm.at[1, pl.ds(base, CHUNK)], idx_buf.at[1])
            pltpu.sync_copy(table_ref.at[idx_buf.at[0]], rows_buf)
            pltpu.sync_copy(rows_buf, acc.at[idx_buf.at[1]], add=True)
        plsc.subcore_barrier()

        pltpu.sync_copy(acc.at[pl.ds(row0, ROWS_PER_SUB)],
                        out_ref.at[pl.ds(row0, ROWS_PER_SUB)])
        plsc.subcore_barrier()

    @pl.when(c == 0)
    def _():
        one_pass(h0, a0)
        one_pass(h1, a1)

    @pl.when(c == 1)
    def _():
        one_pass(h2, a2)
        one_pass(h3, a3)


def _sc_agg(edges, hblks):
    blk = jax.ShapeDtypeStruct((N_NODES, FB), jnp.float32)
    mesh = plsc.VectorSubcoreMesh(core_axis_name="c", subcore_axis_name="s",
                                  num_cores=NC, num_subcores=NS)
    f = pl.kernel(
        _sc_agg_body,
        out_type=(blk, blk, blk, blk),
        mesh=mesh,
        scratch_types=[
            pltpu.VMEM((2, CHUNK), jnp.int32),
            pltpu.VMEM((CHUNK, FB), jnp.float32),
            pltpu.VMEM((ZROWS, FB), jnp.float32),
            pltpu.VMEM_SHARED((N_NODES, FB), jnp.float32),
        ],
        name="sc_gather_segsum",
    )
    return f(edges, *hblks)


def _inproj_body(vn, w, b, o0, o1, o2, o3):
    h = _leaky(jnp.dot(vn[...], w[...],
                       preferred_element_type=jnp.float32) + b[...])
    for p, o in enumerate((o0, o1, o2, o3)):
        o[...] = h[:, p * FB:(p + 1) * FB]


def _inproj(Vnode, W_in, b_in):
    blk = jax.ShapeDtypeStruct((N_NODES, FB), jnp.float32)
    in_dim = Vnode.shape[1]
    return pl.pallas_call(
        _inproj_body,
        grid=(NGRID,),
        in_specs=[
            pl.BlockSpec((NBLK, in_dim), lambda i: (i, 0)),
            pl.BlockSpec((in_dim, HID), lambda i: (0, 0)),
            pl.BlockSpec((1, HID), lambda i: (0, 0)),
        ],
        out_specs=[pl.BlockSpec((NBLK, FB), lambda i: (i, 0))] * NFB,
        out_shape=[blk] * NFB,
        name="tc_inproj",
    )(Vnode, W_in, b_in.reshape(1, HID))


def _layer_body(a0, a1, a2, a3, h0, h1, h2, h3, w, u, o0, o1, o2, o3):
    a = jnp.concatenate([a0[...], a1[...], a2[...], a3[...]], axis=1)
    h = jnp.concatenate([h0[...], h1[...], h2[...], h3[...]], axis=1)
    z = _leaky(jnp.dot(a, w[...], preferred_element_type=jnp.float32)
               + jnp.dot(h, u[...], preferred_element_type=jnp.float32))
    for p, o in enumerate((o0, o1, o2, o3)):
        o[...] = z[:, p * FB:(p + 1) * FB]


def _layer(ablks, hblks, W, U):
    blk = jax.ShapeDtypeStruct((N_NODES, FB), jnp.float32)
    bspec = pl.BlockSpec((NBLK, FB), lambda i: (i, 0))
    wspec = pl.BlockSpec((HID, HID), lambda i: (0, 0))
    return pl.pallas_call(
        _layer_body,
        grid=(NGRID,),
        in_specs=[bspec] * (2 * NFB) + [wspec, wspec],
        out_specs=[bspec] * NFB,
        out_shape=[blk] * NFB,
        name="tc_layer",
    )(*ablks, *hblks, W, U)


def _head_body(h0, h1, h2, h3, y, wout, bout, wc1, bc1, wc2, bc2, x):
    h = jnp.concatenate([h0[...], h1[...], h2[...], h3[...]], axis=1)
    t = jnp.dot(h, wout[...], preferred_element_type=jnp.float32) + bout[...]
    t = t * y[...]
    v = _leaky(jnp.dot(t, wc1[...], preferred_element_type=jnp.float32)
               + bc1[...])
    x[...] = jnp.dot(v, wc2[...], preferred_element_type=jnp.float32) + bc2[...]


def _head(hblks, y, W_out, b_out, Wc1, bc1, Wc2, bc2):
    bspec = pl.BlockSpec((NBLK, FB), lambda i: (i, 0))

    def full(a):
        return pl.BlockSpec(a.shape, lambda i: tuple(0 for _ in a.shape))

    b_out2, bc12, bc22 = b_out.reshape(1, -1), bc1.reshape(1, -1), bc2.reshape(1, -1)
    return pl.pallas_call(
        _head_body,
        grid=(NGRID,),
        in_specs=[bspec] * NFB + [
            pl.BlockSpec((NBLK, 1), lambda i: (i, 0)),
            full(W_out), full(b_out2), full(Wc1), full(bc12),
            full(Wc2), full(bc22),
        ],
        out_specs=pl.BlockSpec((NBLK, 1), lambda i: (i, 0)),
        out_shape=jax.ShapeDtypeStruct((N_NODES, 1), jnp.float32),
        name="tc_head",
    )(*hblks, y, W_out, b_out2, Wc1, bc12, Wc2, bc22)


def kernel(Vnode, Vedge, y, W_in, b_in, Ws, Us, W_out, b_out, Wc1, bc1,
           Wc2, bc2):
    edges = Vedge.astype(jnp.int32)
    hblks = _inproj(Vnode, W_in, b_in)
    for i in range(Ws.shape[0]):
        ablks = _sc_agg(edges, hblks)
        hblks = _layer(ablks, hblks, Ws[i], Us[i])
    return _head(hblks, y, W_out, b_out, Wc1, bc1, Wc2, bc2)


# speedup vs baseline: 3.1648x; 3.1648x over previous
import functools

import jax
import jax.numpy as jnp
from jax import lax
from jax.experimental import pallas as pl
from jax.experimental.pallas import tpu as pltpu
from jax.experimental.pallas import tpu_sc as plsc

N_NODES = 100000
N_EDGES = 3200000
HID = 64
FB = 16
NFB = HID // FB
NC = 2
NS = 16
CHUNK = 128
NCHUNKS = N_EDGES // CHUNK
CH_BASE = NCHUNKS // NS
CH_REM = NCHUNKS - CH_BASE * NS
N_PAD = 100096
ROWS_PER_SUB = N_PAD // NS
ZROWS = 272

NBLK = 2000
NGRID = N_NODES // NBLK


def _leaky(x):
    return jnp.where(x >= 0, x, 0.01 * x)


def _sc_agg_body(src_hbm, dst_hbm, h0, h1, h2, h3, a0, a1, a2, a3,
                 src_buf, dst_buf, rows_buf, zero_buf, acc):
    c = lax.axis_index("c")
    s = lax.axis_index("s")
    start_chunk = s * CH_BASE + jnp.minimum(s, CH_REM)
    count = CH_BASE + jnp.where(s < CH_REM, 1, 0)
    row0 = s * ROWS_PER_SUB

    @pl.loop(0, ZROWS)
    def _(i):
        zero_buf[i, :] = jnp.zeros((FB,), jnp.float32)

    def one_pass(table_ref, out_ref):
        @pl.loop(0, ROWS_PER_SUB // ZROWS)
        def _(i):
            pltpu.sync_copy(zero_buf, acc.at[pl.ds(row0 + i * ZROWS, ZROWS)])
        plsc.subcore_barrier()

        @pl.loop(0, count)
        def _(i):
            base = (start_chunk + i) * CHUNK
            pltpu.sync_copy(src_hbm.at[pl.ds(base, CHUNK)], src_buf)
            pltpu.sync_copy(dst_hbm.at[pl.ds(base, CHUNK)], dst_buf)
            pltpu.sync_copy(table_ref.at[src_buf], rows_buf)
            pltpu.sync_copy(rows_buf, acc.at[dst_buf], add=True)
        plsc.subcore_barrier()

        pltpu.sync_copy(acc.at[pl.ds(row0, ROWS_PER_SUB)],
                        out_ref.at[pl.ds(row0, ROWS_PER_SUB)])
        plsc.subcore_barrier()

    @pl.when(c == 0)
    def _():
        one_pass(h0, a0)
        one_pass(h1, a1)

    @pl.when(c == 1)
    def _():
        one_pass(h2, a2)
        one_pass(h3, a3)


def _sc_agg(src, dst, hblks):
    blk = jax.ShapeDtypeStruct((N_PAD, FB), jnp.float32)
    mesh = plsc.VectorSubcoreMesh(core_axis_name="c", subcore_axis_name="s",
                                  num_cores=NC, num_subcores=NS)
    f = pl.kernel(
        _sc_agg_body,
        out_type=(blk, blk, blk, blk),
        mesh=mesh,
        scratch_types=[
            pltpu.VMEM((CHUNK,), jnp.int32),
            pltpu.VMEM((CHUNK,), jnp.int32),
            pltpu.VMEM((CHUNK, FB), jnp.float32),
            pltpu.VMEM((ZROWS, FB), jnp.float32),
            pltpu.VMEM_SHARED((N_PAD, FB), jnp.float32),
        ],
        compiler_params=pltpu.CompilerParams(use_tc_tiling_on_sc=False),
        name="sc_gather_segsum",
    )
    return f(src, dst, *hblks)


def _inproj_body(vn, w, b, o0, o1, o2, o3):
    h = _leaky(jnp.dot(vn[...], w[...],
                       preferred_element_type=jnp.float32) + b[...])
    for p, o in enumerate((o0, o1, o2, o3)):
        o[...] = h[:, p * FB:(p + 1) * FB]


def _inproj(Vnode, W_in, b_in):
    blk = jax.ShapeDtypeStruct((N_NODES, FB), jnp.float32)
    in_dim = Vnode.shape[1]
    return pl.pallas_call(
        _inproj_body,
        grid=(NGRID,),
        in_specs=[
            pl.BlockSpec((NBLK, in_dim), lambda i: (i, 0)),
            pl.BlockSpec((in_dim, HID), lambda i: (0, 0)),
            pl.BlockSpec((1, HID), lambda i: (0, 0)),
        ],
        out_specs=[pl.BlockSpec((NBLK, FB), lambda i: (i, 0))] * NFB,
        out_shape=[blk] * NFB,
        name="tc_inproj",
    )(Vnode, W_in, b_in.reshape(1, HID))


def _layer_body(a0, a1, a2, a3, h0, h1, h2, h3, w, u, o0, o1, o2, o3):
    a = jnp.concatenate([a0[...], a1[...], a2[...], a3[...]], axis=1)
    h = jnp.concatenate([h0[...], h1[...], h2[...], h3[...]], axis=1)
    z = _leaky(jnp.dot(a, w[...], preferred_element_type=jnp.float32)
               + jnp.dot(h, u[...], preferred_element_type=jnp.float32))
    for p, o in enumerate((o0, o1, o2, o3)):
        o[...] = z[:, p * FB:(p + 1) * FB]


def _layer(ablks, hblks, W, U):
    blk = jax.ShapeDtypeStruct((N_NODES, FB), jnp.float32)
    bspec = pl.BlockSpec((NBLK, FB), lambda i: (i, 0))
    wspec = pl.BlockSpec((HID, HID), lambda i: (0, 0))
    return pl.pallas_call(
        _layer_body,
        grid=(NGRID,),
        in_specs=[bspec] * (2 * NFB) + [wspec, wspec],
        out_specs=[bspec] * NFB,
        out_shape=[blk] * NFB,
        name="tc_layer",
    )(*ablks, *hblks, W, U)


def _head_body(h0, h1, h2, h3, y, wout, bout, wc1, bc1, wc2, bc2, x):
    h = jnp.concatenate([h0[...], h1[...], h2[...], h3[...]], axis=1)
    t = jnp.dot(h, wout[...], preferred_element_type=jnp.float32) + bout[...]
    t = t * y[...]
    v = _leaky(jnp.dot(t, wc1[...], preferred_element_type=jnp.float32)
               + bc1[...])
    x[...] = jnp.dot(v, wc2[...], preferred_element_type=jnp.float32) + bc2[...]


def _head(hblks, y, W_out, b_out, Wc1, bc1, Wc2, bc2):
    bspec = pl.BlockSpec((NBLK, FB), lambda i: (i, 0))

    def full(a):
        return pl.BlockSpec(a.shape, lambda i: tuple(0 for _ in a.shape))

    b_out2, bc12, bc22 = b_out.reshape(1, -1), bc1.reshape(1, -1), bc2.reshape(1, -1)
    return pl.pallas_call(
        _head_body,
        grid=(NGRID,),
        in_specs=[bspec] * NFB + [
            pl.BlockSpec((NBLK, 1), lambda i: (i, 0)),
            full(W_out), full(b_out2), full(Wc1), full(bc12),
            full(Wc2), full(bc22),
        ],
        out_specs=pl.BlockSpec((NBLK, 1), lambda i: (i, 0)),
        out_shape=jax.ShapeDtypeStruct((N_NODES, 1), jnp.float32),
        name="tc_head",
    )(*hblks, y, W_out, b_out2, Wc1, bc12, Wc2, bc22)


def kernel(Vnode, Vedge, y, W_in, b_in, Ws, Us, W_out, b_out, Wc1, bc1,
           Wc2, bc2):
    edges = Vedge.astype(jnp.int32)
    src, dst = edges[0], edges[1]
    hblks = _inproj(Vnode, W_in, b_in)
    for i in range(Ws.shape[0]):
        ablks = _sc_agg(src, dst, hblks)
        hblks = _layer(ablks, hblks, Ws[i], Us[i])
    return _head(hblks, y, W_out, b_out, Wc1, bc1, Wc2, bc2)


# speedup vs baseline: 13.7598x; 4.3477x over previous
import functools

import jax
import jax.numpy as jnp
from jax import lax
from jax.experimental import pallas as pl
from jax.experimental.pallas import tpu as pltpu
from jax.experimental.pallas import tpu_sc as plsc

N_NODES = 100000
N_EDGES = 3200000
HID = 64
FB = 16
NFB = HID // FB
NC = 2
NS = 16
CHUNK = 128
NCHUNKS = N_EDGES // CHUNK
CH_BASE = NCHUNKS // NS
CH_REM = NCHUNKS - CH_BASE * NS
N_PAD = 100096
ROWS_PER_SUB = N_PAD // NS
ZROWS = 272

NBLK = 2000
NGRID = N_NODES // NBLK


def _leaky(x):
    return jnp.where(x >= 0, x, 0.01 * x)


NSLOT = 6
NMAIN = CH_BASE // NSLOT


def _sc_agg_body(src_hbm, dst_hbm, h0, h1, h2, h3, a0, a1, a2, a3,
                 src_buf, dst_buf, rows_buf, zero_buf, acc,
                 isem, gsem, ssem):
    c = lax.axis_index("c")
    s = lax.axis_index("s")
    start_chunk = s * CH_BASE + jnp.minimum(s, CH_REM)
    count = CH_BASE + jnp.where(s < CH_REM, 1, 0)
    row0 = s * ROWS_PER_SUB

    @pl.loop(0, ZROWS)
    def _(i):
        zero_buf[i, :] = jnp.zeros((FB,), jnp.float32)

    def one_pass(table_ref, out_ref):
        def enq_idx(k, ch):
            base = (start_chunk + ch) * CHUNK
            pltpu.async_copy(src_hbm.at[pl.ds(base, CHUNK)], src_buf.at[k],
                             isem.at[k])
            pltpu.async_copy(dst_hbm.at[pl.ds(base, CHUNK)], dst_buf.at[k],
                             isem.at[k])

        def wait_idx(k):
            pltpu.make_async_copy(src_hbm.at[pl.ds(0, CHUNK)], src_buf.at[k],
                                  isem.at[k]).wait()
            pltpu.make_async_copy(dst_hbm.at[pl.ds(0, CHUNK)], dst_buf.at[k],
                                  isem.at[k]).wait()

        def enq_gat(k):
            pltpu.async_copy(table_ref.at[src_buf.at[k]], rows_buf.at[k],
                             gsem.at[k])

        def wait_gat(k):
            pltpu.make_async_copy(table_ref.at[pl.ds(0, CHUNK)],
                                  rows_buf.at[k], gsem.at[k]).wait()

        def enq_sct(k):
            pltpu.async_copy(rows_buf.at[k], acc.at[dst_buf.at[k]],
                             ssem.at[k], add=True)

        def wait_sct(k):
            pltpu.make_async_copy(table_ref.at[pl.ds(0, CHUNK)],
                                  rows_buf.at[k], ssem.at[k]).wait()

        def body(k, ch):
            kL = (k + 4) % NSLOT
            kG = (k + 2) % NSLOT

            @pl.when(ch + 4 < count)
            def _():
                @pl.when(ch >= 2)
                def _():
                    wait_sct(kL)
                enq_idx(kL, ch + 4)

            @pl.when(ch + 2 < count)
            def _():
                wait_idx(kG)
                enq_gat(kG)

            wait_gat(k)
            enq_sct(k)

        @pl.loop(0, ROWS_PER_SUB // ZROWS)
        def _(i):
            pltpu.sync_copy(zero_buf, acc.at[pl.ds(row0 + i * ZROWS, ZROWS)])
        plsc.subcore_barrier()

        for k in range(4):
            enq_idx(k, k)
        for k in range(2):
            wait_idx(k)
            enq_gat(k)

        @pl.loop(0, NMAIN)
        def _(g):
            for k in range(NSLOT):
                body(k, g * NSLOT + k)

        for k in range(NSLOT):
            ch = NMAIN * NSLOT + k

            @pl.when(ch < count)
            def _():
                body(k, ch)

        for k in range(NSLOT):
            wait_sct(k)
        plsc.subcore_barrier()

        pltpu.sync_copy(acc.at[pl.ds(row0, ROWS_PER_SUB)],
                        out_ref.at[pl.ds(row0, ROWS_PER_SUB)])
        plsc.subcore_barrier()

    @pl.when(c == 0)
    def _():
        one_pass(h0, a0)
        one_pass(h1, a1)

    @pl.when(c == 1)
    def _():
        one_pass(h2, a2)
        one_pass(h3, a3)


def _sc_agg(src, dst, hblks):
    blk = jax.ShapeDtypeStruct((N_PAD, FB), jnp.float32)
    mesh = plsc.VectorSubcoreMesh(core_axis_name="c", subcore_axis_name="s",
                                  num_cores=NC, num_subcores=NS)
    f = pl.kernel(
        _sc_agg_body,
        out_type=(blk, blk, blk, blk),
        mesh=mesh,
        scratch_types=[
            pltpu.VMEM((NSLOT, CHUNK), jnp.int32),
            pltpu.VMEM((NSLOT, CHUNK), jnp.int32),
            pltpu.VMEM((NSLOT, CHUNK, FB), jnp.float32),
            pltpu.VMEM((ZROWS, FB), jnp.float32),
            pltpu.VMEM_SHARED((N_PAD, FB), jnp.float32),
            pltpu.SemaphoreType.DMA((NSLOT,)),
            pltpu.SemaphoreType.DMA((NSLOT,)),
            pltpu.SemaphoreType.DMA((NSLOT,)),
        ],
        compiler_params=pltpu.CompilerParams(use_tc_tiling_on_sc=False),
        name="sc_gather_segsum",
    )
    return f(src, dst, *hblks)


def _inproj_body(vn, w, b, o0, o1, o2, o3):
    h = _leaky(jnp.dot(vn[...], w[...],
                       preferred_element_type=jnp.float32) + b[...])
    for p, o in enumerate((o0, o1, o2, o3)):
        o[...] = h[:, p * FB:(p + 1) * FB]


def _inproj(Vnode, W_in, b_in):
    blk = jax.ShapeDtypeStruct((N_NODES, FB), jnp.float32)
    in_dim = Vnode.shape[1]
    return pl.pallas_call(
        _inproj_body,
        grid=(NGRID,),
        in_specs=[
            pl.BlockSpec((NBLK, in_dim), lambda i: (i, 0)),
            pl.BlockSpec((in_dim, HID), lambda i: (0, 0)),
            pl.BlockSpec((1, HID), lambda i: (0, 0)),
        ],
        out_specs=[pl.BlockSpec((NBLK, FB), lambda i: (i, 0))] * NFB,
        out_shape=[blk] * NFB,
        name="tc_inproj",
    )(Vnode, W_in, b_in.reshape(1, HID))


def _layer_body(a0, a1, a2, a3, h0, h1, h2, h3, w, u, o0, o1, o2, o3):
    a = jnp.concatenate([a0[...], a1[...], a2[...], a3[...]], axis=1)
    h = jnp.concatenate([h0[...], h1[...], h2[...], h3[...]], axis=1)
    z = _leaky(jnp.dot(a, w[...], preferred_element_type=jnp.float32)
               + jnp.dot(h, u[...], preferred_element_type=jnp.float32))
    for p, o in enumerate((o0, o1, o2, o3)):
        o[...] = z[:, p * FB:(p + 1) * FB]


def _layer(ablks, hblks, W, U):
    blk = jax.ShapeDtypeStruct((N_NODES, FB), jnp.float32)
    bspec = pl.BlockSpec((NBLK, FB), lambda i: (i, 0))
    wspec = pl.BlockSpec((HID, HID), lambda i: (0, 0))
    return pl.pallas_call(
        _layer_body,
        grid=(NGRID,),
        in_specs=[bspec] * (2 * NFB) + [wspec, wspec],
        out_specs=[bspec] * NFB,
        out_shape=[blk] * NFB,
        name="tc_layer",
    )(*ablks, *hblks, W, U)


def _head_body(h0, h1, h2, h3, y, wout, bout, wc1, bc1, wc2, bc2, x):
    h = jnp.concatenate([h0[...], h1[...], h2[...], h3[...]], axis=1)
    t = jnp.dot(h, wout[...], preferred_element_type=jnp.float32) + bout[...]
    t = t * y[...]
    v = _leaky(jnp.dot(t, wc1[...], preferred_element_type=jnp.float32)
               + bc1[...])
    x[...] = jnp.dot(v, wc2[...], preferred_element_type=jnp.float32) + bc2[...]


def _head(hblks, y, W_out, b_out, Wc1, bc1, Wc2, bc2):
    bspec = pl.BlockSpec((NBLK, FB), lambda i: (i, 0))

    def full(a):
        return pl.BlockSpec(a.shape, lambda i: tuple(0 for _ in a.shape))

    b_out2, bc12, bc22 = b_out.reshape(1, -1), bc1.reshape(1, -1), bc2.reshape(1, -1)
    return pl.pallas_call(
        _head_body,
        grid=(NGRID,),
        in_specs=[bspec] * NFB + [
            pl.BlockSpec((NBLK, 1), lambda i: (i, 0)),
            full(W_out), full(b_out2), full(Wc1), full(bc12),
            full(Wc2), full(bc22),
        ],
        out_specs=pl.BlockSpec((NBLK, 1), lambda i: (i, 0)),
        out_shape=jax.ShapeDtypeStruct((N_NODES, 1), jnp.float32),
        name="tc_head",
    )(*hblks, y, W_out, b_out2, Wc1, bc12, Wc2, bc22)


def kernel(Vnode, Vedge, y, W_in, b_in, Ws, Us, W_out, b_out, Wc1, bc1,
           Wc2, bc2):
    edges = Vedge.astype(jnp.int32)
    src, dst = edges[0], edges[1]
    hblks = _inproj(Vnode, W_in, b_in)
    for i in range(Ws.shape[0]):
        ablks = _sc_agg(src, dst, hblks)
        hblks = _layer(ablks, hblks, Ws[i], Us[i])
    return _head(hblks, y, W_out, b_out, Wc1, bc1, Wc2, bc2)
